# baseline (speedup 1.0000x reference)
import functools

import numpy as np
import jax
import jax.numpy as jnp
from jax import lax
from jax.experimental import pallas as pl
from jax.experimental.pallas import tpu as pltpu
from jax.experimental.pallas import tpu_sc as plsc

_CONNECT = [(11, 12), (12, 13), (13, 14), (14, 15), (13, 25), (25, 26),
            (26, 27), (27, 29), (29, 30), (13, 17), (17, 18), (18, 19),
            (19, 21), (21, 22), (1, 2), (2, 3), (3, 4), (4, 5), (6, 7),
            (7, 8), (8, 9), (9, 10)]
_CHILD = [c for (_, c) in _CONNECT]
_ROOTS = (0, 1, 6, 11)
_IGNORE = (16, 20, 23, 24, 28, 31)
_EQUAL = (13, 19, 22, 13, 27, 30)

_parent = {c: p for (p, c) in _CONNECT}
_bone_of_child = {c: e for e, c in enumerate(_CHILD)}
_eq_map = dict(zip(_IGNORE, _EQUAL))


def _path_and_root(j):
    bones = []
    while j not in _ROOTS:
        bones.append(_bone_of_child[j])
        j = _parent[j]
    return bones, j


_A = np.zeros((22, 32), np.float32)
_RT = np.zeros((32,), np.int64)
for _j in range(32):
    _bones, _r = _path_and_root(_eq_map.get(_j, _j))
    _RT[_j] = _r
    for _e in _bones:
        _A[_e, _j] = 1.0

_W0 = np.zeros((22, 96), np.float32); _W0[:, 0::3] = _A
_W1 = np.zeros((22, 96), np.float32); _W1[:, 1::3] = _A
_W2 = np.zeros((22, 96), np.float32); _W2[:, 2::3] = _A

_C = np.zeros((96, 96), np.float32)
for _j in range(32):
    for _c in range(3):
        _C[3 * int(_RT[_j]) + _c, 3 * _j + _c] = 1.0

_SR = np.zeros((66, 22), np.float32)
_ST = np.zeros((66, 22), np.float32)
_SP = np.zeros((66, 22), np.float32)
for _e in range(22):
    _SR[3 * _e + 0, _e] = 1.0
    _ST[3 * _e + 1, _e] = 1.0
    _SP[3 * _e + 2, _e] = 1.0

_BATCH_PER_BLK = 8
_T = 100
_ROWS_PER_BLK = _BATCH_PER_BLK * _T

_P = np.zeros((_ROWS_PER_BLK, _BATCH_PER_BLK), np.float32)
for _i in range(_ROWS_PER_BLK):
    _P[_i, _i // _T] = 1.0

_INV_PI = 0.31830987334251404
_PI_HI = 3.140625
_PI_LO = float(np.float32(np.pi - 3.140625))
_SINC = (0.999999997000454, -0.16666659969977798, 0.008333097548004268,
         -0.0001981248476825909, 2.612900350327724e-06)
_COSC = (0.9999999998456127, -0.4999999951142109, 0.04166664187638779,
         -0.0013888432330831527, 2.4763766616282726e-05,
         -2.611494974122714e-07)


del _SINC, _COSC, _INV_PI, _PI_HI, _PI_LO

_INV_2PI_T = 0.15915494309189535
_RND_MAGIC_T = 12582912.0
_SIN_T = (6.283185005187988, -41.341617584228516, 81.60091400146484,
          -76.62655639648438, 41.40345001220703, -12.576395988464355)
_COS_T = (0.999999463558197, -19.73903465270996, 64.93061065673828,
          -85.29596710205078, 58.91254806518555, -21.283008575439453)


def _sincos(t):
    u = t * _INV_2PI_T
    nf = jnp.floor(u + 0.5)
    r = u - nf
    r2 = r * r
    s = jnp.float32(_SIN_T[5])
    for k in (4, 3, 2, 1, 0):
        s = s * r2 + jnp.float32(_SIN_T[k])
    s = s * r
    c = jnp.float32(_COS_T[5])
    for k in (4, 3, 2, 1, 0):
        c = c * r2 + jnp.float32(_COS_T[k])
    return s, c


def _tc_body(pred_ref, obs_ref, sr_ref, st_ref, sp_ref, w0_ref, w1_ref,
             w2_ref, c_ref, p_ref, out_ref):
    p = pred_ref[...]
    f32 = jnp.float32
    hi = lax.Precision.HIGHEST
    dot = functools.partial(jnp.dot, preferred_element_type=f32, precision=hi)
    s66, c66 = _sincos(p)
    r = dot(p, sr_ref[...])
    sp_ = dot(s66, sp_ref[...])
    st_ = dot(s66, st_ref[...])
    cp_ = dot(c66, sp_ref[...])
    ct_ = dot(c66, st_ref[...])
    rsp = r * sp_
    x = rsp * ct_
    y = rsp * st_
    z = r * cp_
    oc = dot(obs_ref[...], c_ref[...])
    rep = dot(p_ref[...], oc)
    out = (dot(x, w0_ref[...]) + dot(z, w1_ref[...]) + dot(y, w2_ref[...])
           + rep)
    out_ref[...] = out


def _tc_kernel(observed_pose, pred_pose, interpret=False):
    B, T, D = pred_pose.shape
    pred_flat = pred_pose.reshape(B * T, D)
    obs_last = observed_pose[:, -1, :]
    n_blocks = (B * T) // _ROWS_PER_BLK
    full = lambda shp: pl.BlockSpec(shp, lambda i: (0, 0))
    out = pl.pallas_call(
        _tc_body,
        grid=(n_blocks,),
        in_specs=[
            pl.BlockSpec((_ROWS_PER_BLK, 66), lambda i: (i, 0)),
            pl.BlockSpec((_BATCH_PER_BLK, 96), lambda i: (i, 0)),
            full((66, 22)), full((66, 22)), full((66, 22)),
            full((22, 96)), full((22, 96)), full((22, 96)),
            full((96, 96)), full((_ROWS_PER_BLK, _BATCH_PER_BLK)),
        ],
        out_specs=pl.BlockSpec((_ROWS_PER_BLK, 96), lambda i: (i, 0)),
        out_shape=jax.ShapeDtypeStruct((B * T, 96), jnp.float32),
        interpret=interpret,
    )(pred_flat, obs_last, _SR, _ST, _SP, _W0, _W1, _W2, _C, _P)
    return out.reshape(B, T, 96)


_NC = 2
_NS = 16
_NW = _NC * _NS
_NROWS = 1024 * 100
_RPW = _NROWS // _NW
_SC_CH = 128
_SC_GROUPS = _SC_CH // 16
_SC_NCH = _RPW // _SC_CH
_BPW = 1024 // _NW

_CHAINS = ((11, (0, 1, 2, 3)), (13, (4, 5, 6, 7, 8)),
           (13, (9, 10, 11, 12, 13)), (1, (14, 15, 16, 17)),
           (6, (18, 19, 20, 21)))
_DUP = {13: (16, 24), 19: (20,), 22: (23,), 27: (28,), 30: (31,)}

_INV_2PI = 0.15915494309189535
_RND_MAGIC = 12582912.0
_TWOPI_HI = 6.28125
_TWOPI_LO = 0.0019353071795864769
_SIN_PI = (0.9999999403953552, -0.16666631400585175, 0.008332890458405018,
           -0.00019820756278932095, 2.712799641813035e-06,
           -2.0872652939374348e-08)
_COS_PI = (1.0, -0.49999991059303284, 0.04166652262210846,
           -0.0013887970708310604, 2.4773420591372997e-05,
           -2.711333593197196e-07, 1.7368988469712576e-09)


def _sf(v):
    return jnp.full((16,), v, jnp.float32)


def _sincos2(t):
    u = t * _sf(_INV_2PI)
    magic = _sf(_RND_MAGIC)
    nf = (u + magic) - magic
    r = u - nf
    r2 = r * r
    s = _sf(_SIN_T[5])
    for k in (4, 3, 2, 1, 0):
        s = s * r2 + _sf(_SIN_T[k])
    s = s * r
    c = _sf(_COS_T[5])
    for k in (4, 3, 2, 1, 0):
        c = c * r2 + _sf(_COS_T[k])
    return s, c


_SC_CH = 320
_SC_NCH = _RPW // _SC_CH
_SC_GROUPS = _SC_CH // 16


def _sc_compute_chunk(ci, pred_v, out_v, obs_v, iota):
    @plsc.parallel_loop(0, _SC_GROUPS, unroll=2)
    def group_body(g):
        sixteen = jnp.full((16,), 16, jnp.int32)
        crow = jnp.broadcast_to(g, (16,)) * sixteen + iota
        pbase = crow * jnp.full((16,), 66, jnp.int32)
        obase = crow * jnp.full((16,), 96, jnp.int32)
        lrow = jnp.broadcast_to(ci * _SC_CH, (16,)) + crow
        ob_base = lax.div(lrow, jnp.full((16,), 100, jnp.int32))
        ob_base = ob_base * jnp.full((16,), 96, jnp.int32)

        def _ic(v):
            return jnp.full((16,), v, jnp.int32)

        def put(col, v):
            plsc.store_scatter(out_v, [obase + _ic(col)], v)

        roots = {}
        for rj in (0, 1, 6, 11):
            vals = []
            for comp in range(3):
                v = plsc.load_gather(obs_v, [ob_base + _ic(3 * rj + comp)])
                put(3 * rj + comp, v)
                vals.append(v)
            roots[rj] = vals

        saved13 = None
        for seed, bones in _CHAINS:
            cur = list(saved13 if seed == 13 else roots[seed])
            for e in bones:
                child = _CHILD[e]
                r_ = plsc.load_gather(pred_v, [pbase + _ic(3 * e)])
                th = plsc.load_gather(pred_v, [pbase + _ic(3 * e + 1)])
                ph = plsc.load_gather(pred_v, [pbase + _ic(3 * e + 2)])
                sp_, cp_ = _sincos2(ph)
                st_, ct_ = _sincos2(th)
                rsp = r_ * sp_
                cur = [cur[0] + rsp * ct_, cur[1] + r_ * cp_,
                       cur[2] + rsp * st_]
                for comp in range(3):
                    put(3 * child + comp, cur[comp])
                for d in _DUP.get(child, ()):
                    for comp in range(3):
                        put(3 * d + comp, cur[comp])
                if child == 13:
                    saved13 = list(cur)


def _sc_body(pred_hbm, obs_hbm, out_hbm, obs_v, pred_v0, pred_v1, out_v,
             sem_in0, sem_in1, sem_out):
    cid = lax.axis_index("c")
    sid = lax.axis_index("s")
    wid = sid * _NC + cid
    pltpu.sync_copy(obs_hbm.at[pl.ds(wid * (_BPW * 96), _BPW * 96)], obs_v)
    row0 = wid * _RPW
    iota = lax.iota(jnp.int32, 16)
    pred_bufs = (pred_v0, pred_v1)
    sems_in = (sem_in0, sem_in1)

    def start_in(ci, b):
        r0 = row0 + ci * _SC_CH
        pltpu.async_copy(pred_hbm.at[pl.ds(r0 * 66, _SC_CH * 66)],
                         pred_bufs[b], sems_in[b])

    def wait_in(b):
        pltpu.make_async_copy(pred_hbm.at[pl.ds(0, _SC_CH * 66)],
                              pred_bufs[b], sems_in[b]).wait()

    def do_chunk(ci, b):
        wait_in(b)
        _sc_compute_chunk(ci, pred_bufs[b], out_v, obs_v, iota)
        pltpu.sync_copy(out_v,
                        out_hbm.at[pl.ds((row0 + ci * _SC_CH) * 96,
                                         _SC_CH * 96)])

    start_in(0, 0)

    def pair_body(k, carry):
        c0 = k * 2
        start_in(c0 + 1, 1)
        do_chunk(c0, 0)

        @pl.when(k < (_SC_NCH // 2) - 1)
        def _():
            start_in(c0 + 2, 0)

        do_chunk(c0 + 1, 1)
        return carry

    lax.fori_loop(0, _SC_NCH // 2, pair_body, 0)


def _sc_kernel(observed_pose, pred_pose):
    B, T, D = pred_pose.shape
    pred_flat = pred_pose.reshape(B * T * 66)
    obs_flat = observed_pose[:, -1, :].reshape(B * 96)
    mesh = plsc.VectorSubcoreMesh(core_axis_name="c", subcore_axis_name="s")
    f = pl.kernel(
        _sc_body, mesh=mesh,
        out_type=jax.ShapeDtypeStruct((B * T * 96,), jnp.float32),
        compiler_params=pltpu.CompilerParams(needs_layout_passes=False),
        scratch_types=[
            pltpu.VMEM((_BPW * 96,), jnp.float32),
            pltpu.VMEM((_SC_CH * 66,), jnp.float32),
            pltpu.VMEM((_SC_CH * 66,), jnp.float32),
            pltpu.VMEM((_SC_CH * 96,), jnp.float32),
            pltpu.SemaphoreType.DMA,
            pltpu.SemaphoreType.DMA,
            pltpu.SemaphoreType.DMA,
        ])
    out = f(pred_flat, obs_flat)
    return out.reshape(B, T, 96)


def kernel(observed_pose, pred_pose):
    return _tc_kernel(observed_pose, pred_pose)


# speedup vs baseline: 2.3021x; 2.3021x over previous
import functools

import numpy as np
import jax
import jax.numpy as jnp
from jax import lax
from jax.experimental import pallas as pl
from jax.experimental.pallas import tpu as pltpu
from jax.experimental.pallas import tpu_sc as plsc

_CONNECT = [(11, 12), (12, 13), (13, 14), (14, 15), (13, 25), (25, 26),
            (26, 27), (27, 29), (29, 30), (13, 17), (17, 18), (18, 19),
            (19, 21), (21, 22), (1, 2), (2, 3), (3, 4), (4, 5), (6, 7),
            (7, 8), (8, 9), (9, 10)]
_CHILD = [c for (_, c) in _CONNECT]
_ROOTS = (0, 1, 6, 11)
_IGNORE = (16, 20, 23, 24, 28, 31)
_EQUAL = (13, 19, 22, 13, 27, 30)

_parent = {c: p for (p, c) in _CONNECT}
_bone_of_child = {c: e for e, c in enumerate(_CHILD)}
_eq_map = dict(zip(_IGNORE, _EQUAL))


def _path_and_root(j):
    bones = []
    while j not in _ROOTS:
        bones.append(_bone_of_child[j])
        j = _parent[j]
    return bones, j


_A = np.zeros((22, 32), np.float32)
_RT = np.zeros((32,), np.int64)
for _j in range(32):
    _bones, _r = _path_and_root(_eq_map.get(_j, _j))
    _RT[_j] = _r
    for _e in _bones:
        _A[_e, _j] = 1.0

_W0 = np.zeros((22, 96), np.float32); _W0[:, 0::3] = _A
_W1 = np.zeros((22, 96), np.float32); _W1[:, 1::3] = _A
_W2 = np.zeros((22, 96), np.float32); _W2[:, 2::3] = _A

_C = np.zeros((96, 96), np.float32)
for _j in range(32):
    for _c in range(3):
        _C[3 * int(_RT[_j]) + _c, 3 * _j + _c] = 1.0

_SR = np.zeros((66, 22), np.float32)
_ST = np.zeros((66, 22), np.float32)
_SP = np.zeros((66, 22), np.float32)
for _e in range(22):
    _SR[3 * _e + 0, _e] = 1.0
    _ST[3 * _e + 1, _e] = 1.0
    _SP[3 * _e + 2, _e] = 1.0

_BATCH_PER_BLK = 8
_T = 100
_ROWS_PER_BLK = _BATCH_PER_BLK * _T

_P = np.zeros((_ROWS_PER_BLK, _BATCH_PER_BLK), np.float32)
for _i in range(_ROWS_PER_BLK):
    _P[_i, _i // _T] = 1.0

_INV_PI = 0.31830987334251404
_PI_HI = 3.140625
_PI_LO = float(np.float32(np.pi - 3.140625))
_SINC = (0.999999997000454, -0.16666659969977798, 0.008333097548004268,
         -0.0001981248476825909, 2.612900350327724e-06)
_COSC = (0.9999999998456127, -0.4999999951142109, 0.04166664187638779,
         -0.0013888432330831527, 2.4763766616282726e-05,
         -2.611494974122714e-07)


del _SINC, _COSC, _INV_PI, _PI_HI, _PI_LO

_INV_2PI_T = 0.15915494309189535
_RND_MAGIC_T = 12582912.0
_SIN_T = (6.283185005187988, -41.341617584228516, 81.60091400146484,
          -76.62655639648438, 41.40345001220703, -12.576395988464355)
_COS_T = (0.999999463558197, -19.73903465270996, 64.93061065673828,
          -85.29596710205078, 58.91254806518555, -21.283008575439453)


def _sincos(t):
    u = t * _INV_2PI_T
    nf = jnp.floor(u + 0.5)
    r = u - nf
    r2 = r * r
    s = jnp.float32(_SIN_T[5])
    for k in (4, 3, 2, 1, 0):
        s = s * r2 + jnp.float32(_SIN_T[k])
    s = s * r
    c = jnp.float32(_COS_T[5])
    for k in (4, 3, 2, 1, 0):
        c = c * r2 + jnp.float32(_COS_T[k])
    return s, c


def _tc_body(pred_ref, obs_ref, sr_ref, st_ref, sp_ref, w0_ref, w1_ref,
             w2_ref, c_ref, p_ref, out_ref):
    p = pred_ref[...]
    f32 = jnp.float32
    dot = functools.partial(jnp.dot, preferred_element_type=f32)
    s66, c66 = _sincos(p)
    r = dot(p, sr_ref[...])
    sp_ = dot(s66, sp_ref[...])
    st_ = dot(s66, st_ref[...])
    cp_ = dot(c66, sp_ref[...])
    ct_ = dot(c66, st_ref[...])
    rsp = r * sp_
    x = rsp * ct_
    y = rsp * st_
    z = r * cp_
    oc = dot(obs_ref[...], c_ref[...])
    rep = dot(p_ref[...], oc)
    out = (dot(x, w0_ref[...]) + dot(z, w1_ref[...]) + dot(y, w2_ref[...])
           + rep)
    out_ref[...] = out


def _tc_kernel(observed_pose, pred_pose, interpret=False):
    B, T, D = pred_pose.shape
    pred_flat = pred_pose.reshape(B * T, D)
    obs_last = observed_pose[:, -1, :]
    n_blocks = (B * T) // _ROWS_PER_BLK
    full = lambda shp: pl.BlockSpec(shp, lambda i: (0, 0))
    out = pl.pallas_call(
        _tc_body,
        grid=(n_blocks,),
        in_specs=[
            pl.BlockSpec((_ROWS_PER_BLK, 66), lambda i: (i, 0)),
            pl.BlockSpec((_BATCH_PER_BLK, 96), lambda i: (i, 0)),
            full((66, 22)), full((66, 22)), full((66, 22)),
            full((22, 96)), full((22, 96)), full((22, 96)),
            full((96, 96)), full((_ROWS_PER_BLK, _BATCH_PER_BLK)),
        ],
        out_specs=pl.BlockSpec((_ROWS_PER_BLK, 96), lambda i: (i, 0)),
        out_shape=jax.ShapeDtypeStruct((B * T, 96), jnp.float32),
        interpret=interpret,
    )(pred_flat, obs_last, _SR, _ST, _SP, _W0, _W1, _W2, _C, _P)
    return out.reshape(B, T, 96)


_NC = 2
_NS = 16
_NW = _NC * _NS
_NROWS = 1024 * 100
_RPW = _NROWS // _NW
_SC_CH = 128
_SC_GROUPS = _SC_CH // 16
_SC_NCH = _RPW // _SC_CH
_BPW = 1024 // _NW

_CHAINS = ((11, (0, 1, 2, 3)), (13, (4, 5, 6, 7, 8)),
           (13, (9, 10, 11, 12, 13)), (1, (14, 15, 16, 17)),
           (6, (18, 19, 20, 21)))
_DUP = {13: (16, 24), 19: (20,), 22: (23,), 27: (28,), 30: (31,)}

_INV_2PI = 0.15915494309189535
_RND_MAGIC = 12582912.0
_TWOPI_HI = 6.28125
_TWOPI_LO = 0.0019353071795864769
_SIN_PI = (0.9999999403953552, -0.16666631400585175, 0.008332890458405018,
           -0.00019820756278932095, 2.712799641813035e-06,
           -2.0872652939374348e-08)
_COS_PI = (1.0, -0.49999991059303284, 0.04166652262210846,
           -0.0013887970708310604, 2.4773420591372997e-05,
           -2.711333593197196e-07, 1.7368988469712576e-09)


def _sf(v):
    return jnp.full((16,), v, jnp.float32)


def _sincos2(t):
    u = t * _sf(_INV_2PI)
    magic = _sf(_RND_MAGIC)
    nf = (u + magic) - magic
    r = u - nf
    r2 = r * r
    s = _sf(_SIN_T[5])
    for k in (4, 3, 2, 1, 0):
        s = s * r2 + _sf(_SIN_T[k])
    s = s * r
    c = _sf(_COS_T[5])
    for k in (4, 3, 2, 1, 0):
        c = c * r2 + _sf(_COS_T[k])
    return s, c


_SC_CH = 320
_SC_NCH = _RPW // _SC_CH
_SC_GROUPS = _SC_CH // 16


def _sc_compute_chunk(ci, pred_v, out_v, obs_v, iota):
    @plsc.parallel_loop(0, _SC_GROUPS, unroll=2)
    def group_body(g):
        sixteen = jnp.full((16,), 16, jnp.int32)
        crow = jnp.broadcast_to(g, (16,)) * sixteen + iota
        pbase = crow * jnp.full((16,), 66, jnp.int32)
        obase = crow * jnp.full((16,), 96, jnp.int32)
        lrow = jnp.broadcast_to(ci * _SC_CH, (16,)) + crow
        ob_base = lax.div(lrow, jnp.full((16,), 100, jnp.int32))
        ob_base = ob_base * jnp.full((16,), 96, jnp.int32)

        def _ic(v):
            return jnp.full((16,), v, jnp.int32)

        def put(col, v):
            plsc.store_scatter(out_v, [obase + _ic(col)], v)

        roots = {}
        for rj in (0, 1, 6, 11):
            vals = []
            for comp in range(3):
                v = plsc.load_gather(obs_v, [ob_base + _ic(3 * rj + comp)])
                put(3 * rj + comp, v)
                vals.append(v)
            roots[rj] = vals

        saved13 = None
        for seed, bones in _CHAINS:
            cur = list(saved13 if seed == 13 else roots[seed])
            for e in bones:
                child = _CHILD[e]
                r_ = plsc.load_gather(pred_v, [pbase + _ic(3 * e)])
                th = plsc.load_gather(pred_v, [pbase + _ic(3 * e + 1)])
                ph = plsc.load_gather(pred_v, [pbase + _ic(3 * e + 2)])
                sp_, cp_ = _sincos2(ph)
                st_, ct_ = _sincos2(th)
                rsp = r_ * sp_
                cur = [cur[0] + rsp * ct_, cur[1] + r_ * cp_,
                       cur[2] + rsp * st_]
                for comp in range(3):
                    put(3 * child + comp, cur[comp])
                for d in _DUP.get(child, ()):
                    for comp in range(3):
                        put(3 * d + comp, cur[comp])
                if child == 13:
                    saved13 = list(cur)


def _sc_body(pred_hbm, obs_hbm, out_hbm, obs_v, pred_v0, pred_v1, out_v,
             sem_in0, sem_in1, sem_out):
    cid = lax.axis_index("c")
    sid = lax.axis_index("s")
    wid = sid * _NC + cid
    pltpu.sync_copy(obs_hbm.at[pl.ds(wid * (_BPW * 96), _BPW * 96)], obs_v)
    row0 = wid * _RPW
    iota = lax.iota(jnp.int32, 16)
    pred_bufs = (pred_v0, pred_v1)
    sems_in = (sem_in0, sem_in1)

    def start_in(ci, b):
        r0 = row0 + ci * _SC_CH
        pltpu.async_copy(pred_hbm.at[pl.ds(r0 * 66, _SC_CH * 66)],
                         pred_bufs[b], sems_in[b])

    def wait_in(b):
        pltpu.make_async_copy(pred_hbm.at[pl.ds(0, _SC_CH * 66)],
                              pred_bufs[b], sems_in[b]).wait()

    def do_chunk(ci, b):
        wait_in(b)
        _sc_compute_chunk(ci, pred_bufs[b], out_v, obs_v, iota)
        pltpu.sync_copy(out_v,
                        out_hbm.at[pl.ds((row0 + ci * _SC_CH) * 96,
                                         _SC_CH * 96)])

    start_in(0, 0)

    def pair_body(k, carry):
        c0 = k * 2
        start_in(c0 + 1, 1)
        do_chunk(c0, 0)

        @pl.when(k < (_SC_NCH // 2) - 1)
        def _():
            start_in(c0 + 2, 0)

        do_chunk(c0 + 1, 1)
        return carry

    lax.fori_loop(0, _SC_NCH // 2, pair_body, 0)


def _sc_kernel(observed_pose, pred_pose):
    B, T, D = pred_pose.shape
    pred_flat = pred_pose.reshape(B * T * 66)
    obs_flat = observed_pose[:, -1, :].reshape(B * 96)
    mesh = plsc.VectorSubcoreMesh(core_axis_name="c", subcore_axis_name="s")
    f = pl.kernel(
        _sc_body, mesh=mesh,
        out_type=jax.ShapeDtypeStruct((B * T * 96,), jnp.float32),
        compiler_params=pltpu.CompilerParams(needs_layout_passes=False),
        scratch_types=[
            pltpu.VMEM((_BPW * 96,), jnp.float32),
            pltpu.VMEM((_SC_CH * 66,), jnp.float32),
            pltpu.VMEM((_SC_CH * 66,), jnp.float32),
            pltpu.VMEM((_SC_CH * 96,), jnp.float32),
            pltpu.SemaphoreType.DMA,
            pltpu.SemaphoreType.DMA,
            pltpu.SemaphoreType.DMA,
        ])
    out = f(pred_flat, obs_flat)
    return out.reshape(B, T, 96)


def kernel(observed_pose, pred_pose):
    return _tc_kernel(observed_pose, pred_pose)


# speedup vs baseline: 2.5672x; 1.1152x over previous
import functools

import numpy as np
import jax
import jax.numpy as jnp
from jax import lax
from jax.experimental import pallas as pl
from jax.experimental.pallas import tpu as pltpu
from jax.experimental.pallas import tpu_sc as plsc

_CONNECT = [(11, 12), (12, 13), (13, 14), (14, 15), (13, 25), (25, 26),
            (26, 27), (27, 29), (29, 30), (13, 17), (17, 18), (18, 19),
            (19, 21), (21, 22), (1, 2), (2, 3), (3, 4), (4, 5), (6, 7),
            (7, 8), (8, 9), (9, 10)]
_CHILD = [c for (_, c) in _CONNECT]
_ROOTS = (0, 1, 6, 11)
_IGNORE = (16, 20, 23, 24, 28, 31)
_EQUAL = (13, 19, 22, 13, 27, 30)

_parent = {c: p for (p, c) in _CONNECT}
_bone_of_child = {c: e for e, c in enumerate(_CHILD)}
_eq_map = dict(zip(_IGNORE, _EQUAL))


def _path_and_root(j):
    bones = []
    while j not in _ROOTS:
        bones.append(_bone_of_child[j])
        j = _parent[j]
    return bones, j


_A = np.zeros((22, 32), np.float32)
_RT = np.zeros((32,), np.int64)
for _j in range(32):
    _bones, _r = _path_and_root(_eq_map.get(_j, _j))
    _RT[_j] = _r
    for _e in _bones:
        _A[_e, _j] = 1.0

_W0 = np.zeros((22, 96), np.float32); _W0[:, 0::3] = _A
_W1 = np.zeros((22, 96), np.float32); _W1[:, 1::3] = _A
_W2 = np.zeros((22, 96), np.float32); _W2[:, 2::3] = _A

_C = np.zeros((96, 96), np.float32)
for _j in range(32):
    for _c in range(3):
        _C[3 * int(_RT[_j]) + _c, 3 * _j + _c] = 1.0

_SR = np.zeros((66, 22), np.float32)
_ST = np.zeros((66, 22), np.float32)
_SP = np.zeros((66, 22), np.float32)
for _e in range(22):
    _SR[3 * _e + 0, _e] = 1.0
    _ST[3 * _e + 1, _e] = 1.0
    _SP[3 * _e + 2, _e] = 1.0

_BATCH_PER_BLK = 32
_T = 100
_ROWS_PER_BLK = _BATCH_PER_BLK * _T

_P = np.zeros((_ROWS_PER_BLK, _BATCH_PER_BLK), np.float32)
for _i in range(_ROWS_PER_BLK):
    _P[_i, _i // _T] = 1.0

_INV_PI = 0.31830987334251404
_PI_HI = 3.140625
_PI_LO = float(np.float32(np.pi - 3.140625))
_SINC = (0.999999997000454, -0.16666659969977798, 0.008333097548004268,
         -0.0001981248476825909, 2.612900350327724e-06)
_COSC = (0.9999999998456127, -0.4999999951142109, 0.04166664187638779,
         -0.0013888432330831527, 2.4763766616282726e-05,
         -2.611494974122714e-07)


del _SINC, _COSC, _INV_PI, _PI_HI, _PI_LO

_INV_2PI_T = 0.15915494309189535
_RND_MAGIC_T = 12582912.0
_SIN_T = (6.283185005187988, -41.341617584228516, 81.60091400146484,
          -76.62655639648438, 41.40345001220703, -12.576395988464355)
_COS_T = (0.999999463558197, -19.73903465270996, 64.93061065673828,
          -85.29596710205078, 58.91254806518555, -21.283008575439453)


def _sincos(t):
    u = t * _INV_2PI_T
    nf = jnp.floor(u + 0.5)
    r = u - nf
    r2 = r * r
    s = jnp.float32(_SIN_T[5])
    for k in (4, 3, 2, 1, 0):
        s = s * r2 + jnp.float32(_SIN_T[k])
    s = s * r
    c = jnp.float32(_COS_T[5])
    for k in (4, 3, 2, 1, 0):
        c = c * r2 + jnp.float32(_COS_T[k])
    return s, c


def _tc_body(pred_ref, obs_ref, sr_ref, st_ref, sp_ref, w0_ref, w1_ref,
             w2_ref, c_ref, p_ref, out_ref):
    p = pred_ref[...]
    f32 = jnp.float32
    dot = functools.partial(jnp.dot, preferred_element_type=f32)
    s66, c66 = _sincos(p)
    r = dot(p, sr_ref[...])
    sp_ = dot(s66, sp_ref[...])
    st_ = dot(s66, st_ref[...])
    cp_ = dot(c66, sp_ref[...])
    ct_ = dot(c66, st_ref[...])
    rsp = r * sp_
    x = rsp * ct_
    y = rsp * st_
    z = r * cp_
    oc = dot(obs_ref[...], c_ref[...])
    rep = dot(p_ref[...], oc)
    out = (dot(x, w0_ref[...]) + dot(z, w1_ref[...]) + dot(y, w2_ref[...])
           + rep)
    out_ref[...] = out


def _tc_kernel(observed_pose, pred_pose, interpret=False):
    B, T, D = pred_pose.shape
    pred_flat = pred_pose.reshape(B * T, D)
    obs_last = observed_pose[:, -1, :]
    n_blocks = (B * T) // _ROWS_PER_BLK
    full = lambda shp: pl.BlockSpec(shp, lambda i: (0, 0))
    out = pl.pallas_call(
        _tc_body,
        grid=(n_blocks,),
        in_specs=[
            pl.BlockSpec((_ROWS_PER_BLK, 66), lambda i: (i, 0)),
            pl.BlockSpec((_BATCH_PER_BLK, 96), lambda i: (i, 0)),
            full((66, 22)), full((66, 22)), full((66, 22)),
            full((22, 96)), full((22, 96)), full((22, 96)),
            full((96, 96)), full((_ROWS_PER_BLK, _BATCH_PER_BLK)),
        ],
        out_specs=pl.BlockSpec((_ROWS_PER_BLK, 96), lambda i: (i, 0)),
        out_shape=jax.ShapeDtypeStruct((B * T, 96), jnp.float32),
        interpret=interpret,
    )(pred_flat, obs_last, _SR, _ST, _SP, _W0, _W1, _W2, _C, _P)
    return out.reshape(B, T, 96)


_NC = 2
_NS = 16
_NW = _NC * _NS
_NROWS = 1024 * 100
_RPW = _NROWS // _NW
_SC_CH = 128
_SC_GROUPS = _SC_CH // 16
_SC_NCH = _RPW // _SC_CH
_BPW = 1024 // _NW

_CHAINS = ((11, (0, 1, 2, 3)), (13, (4, 5, 6, 7, 8)),
           (13, (9, 10, 11, 12, 13)), (1, (14, 15, 16, 17)),
           (6, (18, 19, 20, 21)))
_DUP = {13: (16, 24), 19: (20,), 22: (23,), 27: (28,), 30: (31,)}

_INV_2PI = 0.15915494309189535
_RND_MAGIC = 12582912.0
_TWOPI_HI = 6.28125
_TWOPI_LO = 0.0019353071795864769
_SIN_PI = (0.9999999403953552, -0.16666631400585175, 0.008332890458405018,
           -0.00019820756278932095, 2.712799641813035e-06,
           -2.0872652939374348e-08)
_COS_PI = (1.0, -0.49999991059303284, 0.04166652262210846,
           -0.0013887970708310604, 2.4773420591372997e-05,
           -2.711333593197196e-07, 1.7368988469712576e-09)


def _sf(v):
    return jnp.full((16,), v, jnp.float32)


def _sincos2(t):
    u = t * _sf(_INV_2PI)
    magic = _sf(_RND_MAGIC)
    nf = (u + magic) - magic
    r = u - nf
    r2 = r * r
    s = _sf(_SIN_T[5])
    for k in (4, 3, 2, 1, 0):
        s = s * r2 + _sf(_SIN_T[k])
    s = s * r
    c = _sf(_COS_T[5])
    for k in (4, 3, 2, 1, 0):
        c = c * r2 + _sf(_COS_T[k])
    return s, c


_SC_CH = 320
_SC_NCH = _RPW // _SC_CH
_SC_GROUPS = _SC_CH // 16


def _sc_compute_chunk(ci, pred_v, out_v, obs_v, iota):
    @plsc.parallel_loop(0, _SC_GROUPS, unroll=2)
    def group_body(g):
        sixteen = jnp.full((16,), 16, jnp.int32)
        crow = jnp.broadcast_to(g, (16,)) * sixteen + iota
        pbase = crow * jnp.full((16,), 66, jnp.int32)
        obase = crow * jnp.full((16,), 96, jnp.int32)
        lrow = jnp.broadcast_to(ci * _SC_CH, (16,)) + crow
        ob_base = lax.div(lrow, jnp.full((16,), 100, jnp.int32))
        ob_base = ob_base * jnp.full((16,), 96, jnp.int32)

        def _ic(v):
            return jnp.full((16,), v, jnp.int32)

        def put(col, v):
            plsc.store_scatter(out_v, [obase + _ic(col)], v)

        roots = {}
        for rj in (0, 1, 6, 11):
            vals = []
            for comp in range(3):
                v = plsc.load_gather(obs_v, [ob_base + _ic(3 * rj + comp)])
                put(3 * rj + comp, v)
                vals.append(v)
            roots[rj] = vals

        saved13 = None
        for seed, bones in _CHAINS:
            cur = list(saved13 if seed == 13 else roots[seed])
            for e in bones:
                child = _CHILD[e]
                r_ = plsc.load_gather(pred_v, [pbase + _ic(3 * e)])
                th = plsc.load_gather(pred_v, [pbase + _ic(3 * e + 1)])
                ph = plsc.load_gather(pred_v, [pbase + _ic(3 * e + 2)])
                sp_, cp_ = _sincos2(ph)
                st_, ct_ = _sincos2(th)
                rsp = r_ * sp_
                cur = [cur[0] + rsp * ct_, cur[1] + r_ * cp_,
                       cur[2] + rsp * st_]
                for comp in range(3):
                    put(3 * child + comp, cur[comp])
                for d in _DUP.get(child, ()):
                    for comp in range(3):
                        put(3 * d + comp, cur[comp])
                if child == 13:
                    saved13 = list(cur)


def _sc_body(pred_hbm, obs_hbm, out_hbm, obs_v, pred_v0, pred_v1, out_v,
             sem_in0, sem_in1, sem_out):
    cid = lax.axis_index("c")
    sid = lax.axis_index("s")
    wid = sid * _NC + cid
    pltpu.sync_copy(obs_hbm.at[pl.ds(wid * (_BPW * 96), _BPW * 96)], obs_v)
    row0 = wid * _RPW
    iota = lax.iota(jnp.int32, 16)
    pred_bufs = (pred_v0, pred_v1)
    sems_in = (sem_in0, sem_in1)

    def start_in(ci, b):
        r0 = row0 + ci * _SC_CH
        pltpu.async_copy(pred_hbm.at[pl.ds(r0 * 66, _SC_CH * 66)],
                         pred_bufs[b], sems_in[b])

    def wait_in(b):
        pltpu.make_async_copy(pred_hbm.at[pl.ds(0, _SC_CH * 66)],
                              pred_bufs[b], sems_in[b]).wait()

    def do_chunk(ci, b):
        wait_in(b)
        _sc_compute_chunk(ci, pred_bufs[b], out_v, obs_v, iota)
        pltpu.sync_copy(out_v,
                        out_hbm.at[pl.ds((row0 + ci * _SC_CH) * 96,
                                         _SC_CH * 96)])

    start_in(0, 0)

    def pair_body(k, carry):
        c0 = k * 2
        start_in(c0 + 1, 1)
        do_chunk(c0, 0)

        @pl.when(k < (_SC_NCH // 2) - 1)
        def _():
            start_in(c0 + 2, 0)

        do_chunk(c0 + 1, 1)
        return carry

    lax.fori_loop(0, _SC_NCH // 2, pair_body, 0)


def _sc_kernel(observed_pose, pred_pose):
    B, T, D = pred_pose.shape
    pred_flat = pred_pose.reshape(B * T * 66)
    obs_flat = observed_pose[:, -1, :].reshape(B * 96)
    mesh = plsc.VectorSubcoreMesh(core_axis_name="c", subcore_axis_name="s")
    f = pl.kernel(
        _sc_body, mesh=mesh,
        out_type=jax.ShapeDtypeStruct((B * T * 96,), jnp.float32),
        compiler_params=pltpu.CompilerParams(needs_layout_passes=False),
        scratch_types=[
            pltpu.VMEM((_BPW * 96,), jnp.float32),
            pltpu.VMEM((_SC_CH * 66,), jnp.float32),
            pltpu.VMEM((_SC_CH * 66,), jnp.float32),
            pltpu.VMEM((_SC_CH * 96,), jnp.float32),
            pltpu.SemaphoreType.DMA,
            pltpu.SemaphoreType.DMA,
            pltpu.SemaphoreType.DMA,
        ])
    out = f(pred_flat, obs_flat)
    return out.reshape(B, T, 96)


def kernel(observed_pose, pred_pose):
    return _tc_kernel(observed_pose, pred_pose)
